# baseline (speedup 1.0000x reference)
import functools

import jax
import jax.numpy as jnp
from jax import lax
from jax.experimental import pallas as pl
from jax.experimental.pallas import tpu as pltpu
from jax.experimental.pallas import tpu_sc as plsc

EMB = 64
SEQS = 4096
TOK = 200
NC, NS = 2, 16
NW = NC * NS
WSEQ = SEQS // NW
NBUF = 4
IDX_SPLIT = ((0, 104), (104, 96))

_mesh = plsc.VectorSubcoreMesh(
    core_axis_name="c", subcore_axis_name="s", num_cores=NC, num_subcores=NS
)


@functools.partial(
    pl.kernel,
    out_type=jax.ShapeDtypeStruct((SEQS * EMB,), jnp.float32),
    mesh=_mesh,
    compiler_params=pltpu.CompilerParams(
        needs_layout_passes=False, use_tc_tiling_on_sc=False
    ),
    scratch_types=[
        pltpu.VMEM((WSEQ * TOK,), jnp.int32),
        pltpu.VMEM((NBUF, TOK, EMB), jnp.float32),
        pltpu.VMEM((WSEQ * EMB,), jnp.float32),
        pltpu.SemaphoreType.DMA,
        pltpu.SemaphoreType.DMA,
        pltpu.SemaphoreType.DMA,
        pltpu.SemaphoreType.DMA,
    ],
)
def _encode(tok_hbm, table_hbm, out_hbm, tok_v, rows_v, out_v, s0, s1, s2, s3):
    sems = (s0, s1, s2, s3)
    wid = lax.axis_index("s") * NC + lax.axis_index("c")
    tok_base = wid * (WSEQ * TOK)
    pltpu.sync_copy(tok_hbm.at[pl.ds(tok_base, WSEQ * TOK)], tok_v)

    def gather_descrs(c, b):
        ds = []
        for off, n in IDX_SPLIT:
            idx = tok_v.at[pl.ds(c * TOK + off, n)]
            dst = rows_v.at[b].at[pl.ds(off, n)]
            ds.append(pltpu.make_async_copy(table_hbm.at[idx], dst, sems[b]))
        return ds

    def start_gather(c, b):
        for d in gather_descrs(c, b):
            d.start()

    def wait_gather(c, b):
        for d in gather_descrs(c, b):
            d.wait()

    zero = jnp.zeros((16,), jnp.float32)
    lane = lax.iota(jnp.int32, 16)

    def reduce_seq(c, b):
        def body(t, accs):
            return tuple(
                accs[d] + rows_v[b, t, pl.ds(d * 16, 16)] for d in range(4)
            )

        accs = lax.fori_loop(0, TOK, body, (zero,) * 4, unroll=4)

        tbase = c * TOK
        cnt = jnp.zeros((16,), jnp.int32)
        for j in range(12):
            v = tok_v[pl.ds(tbase + j * 16, 16)]
            cnt = cnt + plsc.all_reduce_population_count(v != 0)
        v = tok_v[pl.ds(tbase + 184, 16)]
        cnt = cnt + plsc.all_reduce_population_count((lane >= 8) & (v != 0))

        denom = jnp.maximum(cnt.astype(jnp.float32), 1.0)
        obase = c * EMB
        for d in range(4):
            out_v[pl.ds(obase + d * 16, 16)] = accs[d] / denom

    def step(c, b, last):
        wait_gather(c, b)
        reduce_seq(c, b)
        if not last:
            start_gather(c + NBUF, b)

    for b in range(NBUF):
        start_gather(b, b)

    def loop_body(i, _):
        c = NBUF * i
        for b in range(NBUF):
            step(c + b, b, False)
        return 0

    lax.fori_loop(0, WSEQ // NBUF - 1, loop_body, 0)
    for b in range(NBUF):
        step(WSEQ - NBUF + b, b, True)

    pltpu.sync_copy(out_v, out_hbm.at[pl.ds(wid * (WSEQ * EMB), WSEQ * EMB)])


TAB = 1_000_000
TBLK = 8192


@functools.partial(
    pl.pallas_call,
    grid=((TAB + TBLK - 1) // TBLK,),
    in_specs=[pl.BlockSpec((EMB, TBLK), lambda i: (0, i))],
    out_specs=pl.BlockSpec((TBLK * EMB,), lambda i: (i,)),
    out_shape=jax.ShapeDtypeStruct((TAB * EMB,), jnp.float32),
)
def _linearize(tt_ref, flat_ref):
    flat_ref[...] = tt_ref[...].T.reshape(-1)


def kernel(token_ids, table):
    zero = lax.optimization_barrier(jnp.float32(0.0))
    flat = table.reshape(-1) + zero
    out = _encode(token_ids.reshape(-1), flat.reshape(table.shape))
    return out.reshape(SEQS, EMB)


# speedup vs baseline: 2.7594x; 2.7594x over previous
import functools

import jax
import jax.numpy as jnp
from jax import lax
from jax.experimental import pallas as pl
from jax.experimental.pallas import tpu as pltpu
from jax.experimental.pallas import tpu_sc as plsc

EMB = 64
SEQS = 4096
TOK = 200
NC, NS = 2, 16
NW = NC * NS
WSEQ = SEQS // NW
NBUF = 4
IDX_SPLIT = ((0, 104), (104, 96))

TAB = 1_000_000
TBLK = 8192
HB = TBLK // 2
NTB = (TAB + TBLK - 1) // TBLK

_mesh = plsc.VectorSubcoreMesh(
    core_axis_name="c", subcore_axis_name="s", num_cores=NC, num_subcores=NS
)


@functools.partial(
    pl.pallas_call,
    grid=(NTB,),
    in_specs=[pl.BlockSpec((EMB, TBLK), lambda i: (0, i))],
    out_specs=pl.BlockSpec((TBLK * EMB,), lambda i: (i,)),
    out_shape=jax.ShapeDtypeStruct((NTB * TBLK * EMB,), jnp.float32),
)
def _linearize(tt_ref, flat_ref):
    x = tt_ref[...]
    z = jnp.concatenate([x[:, :HB].T, x[:, HB:].T], axis=1)
    flat_ref[...] = z.reshape(-1)


@functools.partial(
    pl.kernel,
    out_type=jax.ShapeDtypeStruct((SEQS * EMB,), jnp.float32),
    mesh=_mesh,
    compiler_params=pltpu.CompilerParams(
        needs_layout_passes=False, use_tc_tiling_on_sc=False
    ),
    scratch_types=[
        pltpu.VMEM((WSEQ * TOK,), jnp.int32),
        pltpu.VMEM((NBUF, TOK), jnp.int32),
        pltpu.VMEM((NBUF, TOK, EMB), jnp.float32),
        pltpu.VMEM((WSEQ * EMB,), jnp.float32),
        pltpu.SemaphoreType.DMA,
        pltpu.SemaphoreType.DMA,
        pltpu.SemaphoreType.DMA,
        pltpu.SemaphoreType.DMA,
    ],
)
def _encode(
    tok_hbm, packed_hbm, out_hbm, tok_v, pidx_v, rows_v, out_v, s0, s1, s2, s3
):
    sems = (s0, s1, s2, s3)
    wid = lax.axis_index("s") * NC + lax.axis_index("c")
    tok_base = wid * (WSEQ * TOK)
    pltpu.sync_copy(tok_hbm.at[pl.ds(tok_base, WSEQ * TOK)], tok_v)

    def gather_descrs(b):
        ds = []
        for off, n in IDX_SPLIT:
            idx = pidx_v.at[b].at[pl.ds(off, n)]
            dst = rows_v.at[b].at[pl.ds(off, n)]
            ds.append(pltpu.make_async_copy(packed_hbm.at[idx], dst, sems[b]))
        return ds

    def start_gather(c, b):
        tbase = c * TOK
        for off in tuple(j * 16 for j in range(12)) + (184,):
            v = tok_v[pl.ds(tbase + off, 16)]
            q = ((v >> 13) << 13) | ((v & 4095) << 1) | ((v >> 12) & 1)
            pidx_v[b, pl.ds(off, 16)] = q
        for d in gather_descrs(b):
            d.start()

    def wait_gather(b):
        for d in gather_descrs(b):
            d.wait()

    zero = jnp.zeros((16,), jnp.float32)
    lane = lax.iota(jnp.int32, 16)

    def reduce_seq(c, b):
        def body(t, accs):
            return tuple(
                accs[d] + rows_v[b, t, pl.ds(d * 16, 16)] for d in range(4)
            )

        accs = lax.fori_loop(0, TOK, body, (zero,) * 4, unroll=4)

        tbase = c * TOK
        cnt = jnp.zeros((16,), jnp.int32)
        for j in range(12):
            v = tok_v[pl.ds(tbase + j * 16, 16)]
            cnt = cnt + plsc.all_reduce_population_count(v != 0)
        v = tok_v[pl.ds(tbase + 184, 16)]
        cnt = cnt + plsc.all_reduce_population_count((lane >= 8) & (v != 0))

        denom = jnp.maximum(cnt.astype(jnp.float32), 1.0)
        obase = c * EMB
        for d in range(4):
            out_v[pl.ds(obase + d * 16, 16)] = accs[d] / denom

    def step(c, b, last):
        wait_gather(b)
        reduce_seq(c, b)
        if not last:
            start_gather(c + NBUF, b)

    for b in range(NBUF):
        start_gather(b, b)

    def loop_body(i, _):
        c = NBUF * i
        for b in range(NBUF):
            step(c + b, b, False)
        return 0

    lax.fori_loop(0, WSEQ // NBUF - 1, loop_body, 0)
    for b in range(NBUF):
        step(WSEQ - NBUF + b, b, True)

    pltpu.sync_copy(out_v, out_hbm.at[pl.ds(wid * (WSEQ * EMB), WSEQ * EMB)])


def kernel(token_ids, table):
    packed = _linearize(table.T).reshape(NTB * TBLK, EMB)
    out = _encode(token_ids.reshape(-1), packed)
    return out.reshape(SEQS, EMB)
